# baseline (speedup 1.0000x reference)
import functools

import jax
import jax.numpy as jnp
from jax import lax
from jax.experimental import pallas as pl
from jax.experimental.pallas import tpu as pltpu
from jax.experimental.pallas import tpu_sc as plsc

_N = 10000
_E = 320000
_F = 128
_NW = 32
_EPW = _E // _NW
_K = 80
_CH = _EPW // _K
_RPT = 624
_REM = _N - 16 * _RPT


def _sc_agg_body(h_hbm, src_hbm, dst_hbm, w_hbm, out0, out1,
                 dst_v, sbuf0, sbuf1, sbuf2, wbuf0, wbuf1, wbuf2,
                 rows0, rows1, rows2, agg_sh,
                 gsem0, gsem1, gsem2, ssem0, ssem1, ssem2,
                 wsem0, wsem1, wsem2, xsem0, xsem1, xsem2):
    cid = lax.axis_index("c")
    sid = lax.axis_index("s")
    wid = sid * 2 + cid

    zeros = jnp.zeros((16,), jnp.float32)

    def zrow(i, carry):
        for j in range(8):
            rows0[i, pl.ds(j * 16, 16)] = zeros
        return carry

    lax.fori_loop(0, _K, zrow, 0)
    base = sid * _RPT
    for r in range(_RPT // _K):
        pltpu.sync_copy(rows0, agg_sh.at[pl.ds(base + r * _K, _K)])
    rem = _RPT % _K
    pltpu.sync_copy(rows0.at[pl.ds(0, rem)],
                    agg_sh.at[pl.ds(base + (_RPT // _K) * _K, rem)])

    @pl.when(sid == 0)
    def _():
        pltpu.sync_copy(rows0.at[pl.ds(0, _REM)],
                        agg_sh.at[pl.ds(16 * _RPT, _REM)])

    pltpu.sync_copy(dst_hbm.at[wid], dst_v)
    plsc.subcore_barrier()

    bufs = (rows0, rows1, rows2)
    gsems = (gsem0, gsem1, gsem2)
    ssems = (ssem0, ssem1, ssem2)
    wbufs = (wbuf0, wbuf1, wbuf2)
    wsems = (wsem0, wsem1, wsem2)
    sbufs = (sbuf0, sbuf1, sbuf2)
    xsems = (xsem0, xsem1, xsem2)

    def w_start(c, b):
        pltpu.async_copy(
            w_hbm.at[pl.ds(wid * _EPW + c * _K, _K)], wbufs[b], wsems[b])

    def w_wait(c, b):
        pltpu.make_async_copy(
            w_hbm.at[pl.ds(wid * _EPW + c * _K, _K)],
            wbufs[b], wsems[b]).wait()

    def x_start(c, b):
        pltpu.async_copy(
            src_hbm.at[pl.ds(wid * _EPW + c * _K, _K)], sbufs[b], xsems[b])

    def x_wait(c, b):
        pltpu.make_async_copy(
            src_hbm.at[pl.ds(wid * _EPW + c * _K, _K)],
            sbufs[b], xsems[b]).wait()

    def g_start(c, b):
        pltpu.async_copy(h_hbm.at[sbufs[b]], bufs[b], gsems[b])

    def g_wait(c, b):
        pltpu.make_async_copy(h_hbm.at[sbufs[b]], bufs[b], gsems[b]).wait()

    def s_start(c, b):
        pltpu.async_copy(bufs[b], agg_sh.at[dst_v.at[c]], ssems[b], add=True)

    def s_wait(c, b):
        pltpu.make_async_copy(bufs[b], agg_sh.at[dst_v.at[c]],
                              ssems[b]).wait()

    def mult(c, b):
        buf, wb = bufs[b], wbufs[b]

        def group(g, c2):
            w16 = wb[pl.ds(g * 16, 16)]
            for l in range(16):
                wsp = jnp.full((16,), w16[l])
                e = g * 16 + l
                for j in range(8):
                    sl = pl.ds(j * 16, 16)
                    buf[e, sl] = buf[e, sl] * wsp
            return c2

        lax.fori_loop(0, _K // 16, group, 0, unroll=True)

    x_start(0, 0)
    x_start(1, 1)
    x_start(2, 2)
    w_start(0, 0)
    w_start(1, 1)
    w_start(2, 2)
    x_wait(0, 0)
    g_start(0, 0)

    def do_chunk(c, b):
        bn = (b + 1) % 3

        @pl.when(c >= 2)
        def _():
            s_wait(c - 2, bn)

        @pl.when(c + 1 < _CH)
        def _():
            x_wait(c + 1, bn)
            g_start(c + 1, bn)

        g_wait(c, b)

        @pl.when(c + 3 < _CH)
        def _():
            x_start(c + 3, b)

        w_wait(c, b)
        mult(c, b)
        s_start(c, b)

        @pl.when(c + 3 < _CH)
        def _():
            w_start(c + 3, b)

    def step3(i, carry):
        c0 = i * 3
        for b in range(3):
            do_chunk(c0 + b, b)
        return carry

    nfull = (_CH // 3) * 3
    lax.fori_loop(0, _CH // 3, step3, 0)
    for c in range(nfull, _CH):
        do_chunk(c, c % 3)
    s_wait(_CH - 2, (_CH - 2) % 3)
    s_wait(_CH - 1, (_CH - 1) % 3)
    plsc.subcore_barrier()

    @pl.when(cid == 0)
    def _():
        pltpu.sync_copy(agg_sh.at[pl.ds(base, _RPT)],
                        out0.at[pl.ds(base, _RPT)])

        @pl.when(sid == 0)
        def _():
            pltpu.sync_copy(agg_sh.at[pl.ds(16 * _RPT, _REM)],
                            out0.at[pl.ds(16 * _RPT, _REM)])

    @pl.when(cid == 1)
    def _():
        pltpu.sync_copy(agg_sh.at[pl.ds(base, _RPT)],
                        out1.at[pl.ds(base, _RPT)])

        @pl.when(sid == 0)
        def _():
            pltpu.sync_copy(agg_sh.at[pl.ds(16 * _RPT, _REM)],
                            out1.at[pl.ds(16 * _RPT, _REM)])


def _sc_agg(h, src2, dst2, w):
    mesh = plsc.VectorSubcoreMesh(core_axis_name="c", subcore_axis_name="s")
    f = pl.kernel(
        _sc_agg_body,
        out_type=[jax.ShapeDtypeStruct((_N, _F), jnp.float32)] * 2,
        mesh=mesh,
        scratch_types=(
            [pltpu.VMEM((_CH, _K), jnp.int32)]
            + [pltpu.VMEM((_K,), jnp.int32)] * 3
            + [pltpu.VMEM((_K,), jnp.float32)] * 3
            + [pltpu.VMEM((_K, _F), jnp.float32)] * 3
            + [pltpu.VMEM_SHARED((_N, _F), jnp.float32)]
            + [pltpu.SemaphoreType.DMA] * 12
        ),
    )
    return f(h, src2, dst2, w)


def _mm_body(a_ref, b_ref, h_ref, wr_ref, wo_ref, bias_ref, o_ref, *, relu):
    agg = a_ref[...] + b_ref[...]
    acc = jnp.dot(agg, wr_ref[...], preferred_element_type=jnp.float32)
    acc = acc + jnp.dot(h_ref[...], wo_ref[...],
                        preferred_element_type=jnp.float32)
    acc = acc + bias_ref[...]
    o_ref[...] = jnp.maximum(acc, 0.0) if relu else acc


def _mm(a, b, h, wr, wo, bias, relu):
    br = 1000
    return pl.pallas_call(
        functools.partial(_mm_body, relu=relu),
        grid=(_N // br,),
        in_specs=[
            pl.BlockSpec((br, _F), lambda i: (i, 0)),
            pl.BlockSpec((br, _F), lambda i: (i, 0)),
            pl.BlockSpec((br, _F), lambda i: (i, 0)),
            pl.BlockSpec((_F, _F), lambda i: (0, 0)),
            pl.BlockSpec((_F, _F), lambda i: (0, 0)),
            pl.BlockSpec((1, _F), lambda i: (0, 0)),
        ],
        out_specs=pl.BlockSpec((br, _F), lambda i: (i, 0)),
        out_shape=jax.ShapeDtypeStruct((_N, _F), jnp.float32),
    )(a, b, h, wr, wo, bias)


def kernel(x, edge_index, edge_weight,
           W1_rel, b1, W1_root, W2_rel, b2, W2_root, W3_rel, b3, W3_root):
    src2 = edge_index[0]
    dst2 = edge_index[1].reshape(_NW, _CH, _K)
    h = x
    layers = [(W1_rel, b1, W1_root, True),
              (W2_rel, b2, W2_root, True),
              (W3_rel, b3, W3_root, False)]
    for wr, b, wo, relu in layers:
        p0, p1 = _sc_agg(h, src2, dst2, edge_weight)
        h = _mm(p0, p1, h, wr, wo, b.reshape(1, _F), relu)
    return h


# speedup vs baseline: 1.3209x; 1.3209x over previous
import functools

import jax
import jax.numpy as jnp
from jax import lax
from jax.experimental import pallas as pl
from jax.experimental.pallas import tpu as pltpu
from jax.experimental.pallas import tpu_sc as plsc

_N = 10000
_E = 320000
_F = 128
_NW = 32
_EPW = _E // _NW
_K = 80
_CH = _EPW // _K
_RPT = 624
_REM = _N - 16 * _RPT


def _sc_agg_body(h_hbm, src_hbm, dst_hbm, w_hbm, out0, out1,
                 dst_v, sbuf0, sbuf1, sbuf2, wbuf0, wbuf1, wbuf2,
                 rows0, rows1, rows2, agg_sh,
                 gsem0, gsem1, gsem2, ssem0, ssem1, ssem2,
                 wsem0, wsem1, wsem2, xsem0, xsem1, xsem2):
    cid = lax.axis_index("c")
    sid = lax.axis_index("s")
    wid = sid * 2 + cid

    zeros = jnp.zeros((16,), jnp.float32)

    def zrow(i, carry):
        for j in range(8):
            rows0[i, pl.ds(j * 16, 16)] = zeros
        return carry

    lax.fori_loop(0, _K, zrow, 0)
    base = sid * _RPT
    for r in range(_RPT // _K):
        pltpu.sync_copy(rows0, agg_sh.at[pl.ds(base + r * _K, _K)])
    rem = _RPT % _K
    pltpu.sync_copy(rows0.at[pl.ds(0, rem)],
                    agg_sh.at[pl.ds(base + (_RPT // _K) * _K, rem)])

    @pl.when(sid == 0)
    def _():
        pltpu.sync_copy(rows0.at[pl.ds(0, _REM)],
                        agg_sh.at[pl.ds(16 * _RPT, _REM)])

    pltpu.sync_copy(dst_hbm.at[wid], dst_v)
    plsc.subcore_barrier()

    bufs = (rows0, rows1, rows2)
    gsems = (gsem0, gsem1, gsem2)
    ssems = (ssem0, ssem1, ssem2)
    wbufs = (wbuf0, wbuf1, wbuf2)
    wsems = (wsem0, wsem1, wsem2)
    sbufs = (sbuf0, sbuf1, sbuf2)
    xsems = (xsem0, xsem1, xsem2)

    def w_start(c, b):
        pltpu.async_copy(
            w_hbm.at[pl.ds(wid * _EPW + c * _K, _K)], wbufs[b], wsems[b])

    def w_wait(c, b):
        pltpu.make_async_copy(
            w_hbm.at[pl.ds(wid * _EPW + c * _K, _K)],
            wbufs[b], wsems[b]).wait()

    def x_start(c, b):
        pltpu.async_copy(
            src_hbm.at[pl.ds(wid * _EPW + c * _K, _K)], sbufs[b], xsems[b])

    def x_wait(c, b):
        pltpu.make_async_copy(
            src_hbm.at[pl.ds(wid * _EPW + c * _K, _K)],
            sbufs[b], xsems[b]).wait()

    def g_start(c, b):
        pltpu.async_copy(h_hbm.at[sbufs[b]], bufs[b], gsems[b])

    def g_wait(c, b):
        pltpu.make_async_copy(h_hbm.at[sbufs[b]], bufs[b], gsems[b]).wait()

    def s_start(c, b):
        pltpu.async_copy(bufs[b], agg_sh.at[dst_v.at[c]], ssems[b], add=True)

    def s_wait(c, b):
        pltpu.make_async_copy(bufs[b], agg_sh.at[dst_v.at[c]],
                              ssems[b]).wait()

    def mult(c, b):
        buf, wb = bufs[b], wbufs[b]

        def group(g, c2):
            w16 = wb[pl.ds(g * 16, 16)]
            for l in range(16):
                wsp = jnp.full((16,), w16[l])
                e = g * 16 + l
                for j in range(8):
                    sl = pl.ds(j * 16, 16)
                    buf[e, sl] = buf[e, sl] * wsp
            return c2

        lax.fori_loop(0, _K // 16, group, 0)

    x_start(0, 0)
    x_start(1, 1)
    x_start(2, 2)
    w_start(0, 0)
    w_start(1, 1)
    w_start(2, 2)
    x_wait(0, 0)
    g_start(0, 0)

    def do_chunk(c, b):
        bn = (b + 1) % 3

        @pl.when(c >= 2)
        def _():
            s_wait(c - 2, bn)

        @pl.when(c + 1 < _CH)
        def _():
            x_wait(c + 1, bn)
            g_start(c + 1, bn)

        g_wait(c, b)

        @pl.when(c + 3 < _CH)
        def _():
            x_start(c + 3, b)

        w_wait(c, b)
        mult(c, b)
        s_start(c, b)

        @pl.when(c + 3 < _CH)
        def _():
            w_start(c + 3, b)

    def step3(i, carry):
        c0 = i * 3
        for b in range(3):
            do_chunk(c0 + b, b)
        return carry

    nfull = (_CH // 3) * 3
    lax.fori_loop(0, _CH // 3, step3, 0)
    for c in range(nfull, _CH):
        do_chunk(c, c % 3)
    s_wait(_CH - 2, (_CH - 2) % 3)
    s_wait(_CH - 1, (_CH - 1) % 3)
    plsc.subcore_barrier()

    @pl.when(cid == 0)
    def _():
        pltpu.sync_copy(agg_sh.at[pl.ds(base, _RPT)],
                        out0.at[pl.ds(base, _RPT)])

        @pl.when(sid == 0)
        def _():
            pltpu.sync_copy(agg_sh.at[pl.ds(16 * _RPT, _REM)],
                            out0.at[pl.ds(16 * _RPT, _REM)])

    @pl.when(cid == 1)
    def _():
        pltpu.sync_copy(agg_sh.at[pl.ds(base, _RPT)],
                        out1.at[pl.ds(base, _RPT)])

        @pl.when(sid == 0)
        def _():
            pltpu.sync_copy(agg_sh.at[pl.ds(16 * _RPT, _REM)],
                            out1.at[pl.ds(16 * _RPT, _REM)])


def _sc_agg(h, src2, dst2, w):
    mesh = plsc.VectorSubcoreMesh(core_axis_name="c", subcore_axis_name="s")
    f = pl.kernel(
        _sc_agg_body,
        out_type=[jax.ShapeDtypeStruct((_N, _F), jnp.float32)] * 2,
        mesh=mesh,
        scratch_types=(
            [pltpu.VMEM((_CH, _K), jnp.int32)]
            + [pltpu.VMEM((_K,), jnp.int32)] * 3
            + [pltpu.VMEM((_K,), jnp.float32)] * 3
            + [pltpu.VMEM((_K, _F), jnp.float32)] * 3
            + [pltpu.VMEM_SHARED((_N, _F), jnp.float32)]
            + [pltpu.SemaphoreType.DMA] * 12
        ),
    )
    return f(h, src2, dst2, w)


def _mm_body(a_ref, b_ref, h_ref, wr_ref, wo_ref, bias_ref, o_ref, *, relu):
    agg = a_ref[...] + b_ref[...]
    acc = jnp.dot(agg, wr_ref[...], preferred_element_type=jnp.float32)
    acc = acc + jnp.dot(h_ref[...], wo_ref[...],
                        preferred_element_type=jnp.float32)
    acc = acc + bias_ref[...]
    o_ref[...] = jnp.maximum(acc, 0.0) if relu else acc


def _mm(a, b, h, wr, wo, bias, relu):
    br = 1000
    return pl.pallas_call(
        functools.partial(_mm_body, relu=relu),
        grid=(_N // br,),
        in_specs=[
            pl.BlockSpec((br, _F), lambda i: (i, 0)),
            pl.BlockSpec((br, _F), lambda i: (i, 0)),
            pl.BlockSpec((br, _F), lambda i: (i, 0)),
            pl.BlockSpec((_F, _F), lambda i: (0, 0)),
            pl.BlockSpec((_F, _F), lambda i: (0, 0)),
            pl.BlockSpec((1, _F), lambda i: (0, 0)),
        ],
        out_specs=pl.BlockSpec((br, _F), lambda i: (i, 0)),
        out_shape=jax.ShapeDtypeStruct((_N, _F), jnp.float32),
    )(a, b, h, wr, wo, bias)


def kernel(x, edge_index, edge_weight,
           W1_rel, b1, W1_root, W2_rel, b2, W2_root, W3_rel, b3, W3_root):
    src2 = edge_index[0]
    dst2 = edge_index[1].reshape(_NW, _CH, _K)
    h = x
    layers = [(W1_rel, b1, W1_root, True),
              (W2_rel, b2, W2_root, True),
              (W3_rel, b3, W3_root, False)]
    for wr, b, wo, relu in layers:
        p0, p1 = _sc_agg(h, src2, dst2, edge_weight)
        h = _mm(p0, p1, h, wr, wo, b.reshape(1, _F), relu)
    return h
